# baseline (speedup 1.0000x reference)
import functools

import jax
import jax.numpy as jnp
from jax import lax
from jax.experimental import pallas as pl
from jax.experimental.pallas import tpu as pltpu
from jax.experimental.pallas import tpu_sc as plsc

_H = 64
_D = 128
_NOUT = 10
_K = 56
_NW = 32
_ROWS = 1000


def _proj_body(x_ref, w_ref, t_ref):
    x = x_ref[...]
    for q in range(5):
        t_ref[q, :, :] = jnp.dot(x, w_ref[q, :, :],
                                 preferred_element_type=jnp.float32)


def _project(x_atom, w):
    n = x_atom.shape[0]
    grid = n // _ROWS
    return pl.pallas_call(
        _proj_body,
        grid=(grid,),
        in_specs=[pl.BlockSpec((_ROWS, _D), lambda i: (i, 0)),
                  pl.BlockSpec((5, _D, _D), lambda i: (0, 0, 0))],
        out_specs=pl.BlockSpec((5, _ROWS, _D), lambda i: (0, i, 0)),
        out_shape=jax.ShapeDtypeStruct((5, n, _D), jnp.float32),
    )(x_atom, w)


def _sc_gather(npad, n_tab, nbuf, kk, valu_row_factory):
    c_per_w = npad // _NW
    nchunk = c_per_w // kk
    n_grp = nchunk // nbuf
    nc = plsc.get_sparse_core_info().num_cores
    seg = kk * n_tab

    scratch = ([pltpu.VMEM((c_per_w * n_tab,), jnp.int32)]
               + [pltpu.VMEM((seg, _D), jnp.float32)] * nbuf
               + [pltpu.VMEM((kk, _D), jnp.float32)] * nbuf
               + [pltpu.SemaphoreType.DMA] * (2 * nbuf))

    @functools.partial(
        pl.kernel, mesh=plsc.VectorSubcoreMesh(core_axis_name="c",
                                               subcore_axis_name="s"),
        out_type=jax.ShapeDtypeStruct((npad, _D), jnp.float32),
        scratch_types=scratch)
    def k(table, cidx_h, out_h, *scr):
        i_all = scr[0]
        r_v = scr[1:1 + nbuf]
        ov_v = scr[1 + nbuf:1 + 2 * nbuf]
        gsem = scr[1 + 2 * nbuf:1 + 3 * nbuf]
        osem = scr[1 + 3 * nbuf:1 + 4 * nbuf]
        wid = lax.axis_index("s") * nc + lax.axis_index("c")
        base = wid * c_per_w
        valu_row = valu_row_factory(r_v, ov_v, kk)

        pltpu.sync_copy(cidx_h.at[pl.ds(base * n_tab, c_per_w * n_tab)],
                        i_all)

        def issue(c, slot):
            for q in range(n_tab):
                pltpu.async_copy(
                    table.at[i_all.at[pl.ds(c * seg + q * kk, kk)]],
                    r_v[slot].at[pl.ds(q * kk, kk)], gsem[slot])

        def body(g, carry):
            for slot in range(nbuf):
                c = nbuf * g + slot
                off = base + c * kk
                for q in range(n_tab):
                    pltpu.make_async_copy(
                        table.at[i_all.at[pl.ds(c * seg + q * kk, kk)]],
                        r_v[slot].at[pl.ds(q * kk, kk)], gsem[slot]).wait()

                @pl.when(g > 0)
                def _():
                    pltpu.make_async_copy(ov_v[slot],
                                          out_h.at[pl.ds(off, kk)],
                                          osem[slot]).wait()

                def row(i, carry2):
                    valu_row(i, slot)
                    return carry2

                lax.fori_loop(0, kk, row, 0, unroll=4)
                pltpu.async_copy(ov_v[slot], out_h.at[pl.ds(off, kk)],
                                 osem[slot])

                @pl.when(g < n_grp - 1)
                def _():
                    issue(c + nbuf, slot)
            return carry

        for slot in range(nbuf):
            issue(slot, slot)
        lax.fori_loop(0, n_grp, body, 0)
        for slot in range(nbuf):
            pltpu.make_async_copy(ov_v[slot], out_h.at[pl.ds(base, kk)],
                                  osem[slot]).wait()

    return k


def _bond_rows(r_v, ov_v, kk):
    def valu_row(i, slot):
        q = r_v[slot]
        ov = ov_v[slot]
        for j in range(4):
            lo = pl.ds(j * 16, 16)
            hi = pl.ds(_H + j * 16, 16)
            ov[i, lo] = q[i, lo] + q[kk + i, hi]
            ov[i, hi] = q[kk + i, lo] + q[i, hi]
    return valu_row


def _angle_rows(r_v, ov_v, kk):
    def valu_row(i, slot):
        q = r_v[slot]
        ov = ov_v[slot]
        for j in range(4):
            lo = pl.ds(j * 16, 16)
            hi = pl.ds(_H + j * 16, 16)
            mid = q[kk + i, lo]
            ov[i, lo] = q[i, lo] + mid + q[2 * kk + i, hi]
            ov[i, hi] = q[2 * kk + i, lo] + mid + q[i, hi]
    return valu_row


def _torsion_rows(r_v, ov_v, kk):
    def valu_row(i, slot):
        q = r_v[slot]
        ov = ov_v[slot]
        for j in range(4):
            lo = pl.ds(j * 16, 16)
            hi = pl.ds(_H + j * 16, 16)
            g = ((q[i, lo] + q[kk + i, hi])
                 + (q[2 * kk + i, lo] + q[3 * kk + i, hi]))
            ov[i, lo] = g
            ov[i, hi] = g
    return valu_row


def _mlp_body(bg, ag, tg, br, ar, tr,
              bw1, bb1, bw2, bb2, bw3, bb3,
              aw1, ab1, aw2, ab2, aw3, ab3,
              tw1, tb1, tw2, tb2, tw3, tb3,
              bo, ao, to):
    def tail(g, rep, w1, b1, w2, b2, w3, b3):
        h = jax.nn.relu(g[...] + rep[...] * w1[...] + b1[...])
        h = jax.nn.relu(jnp.dot(h, w2[...], preferred_element_type=jnp.float32)
                        + b2[...])
        return jnp.dot(h, w3[...], preferred_element_type=jnp.float32) + b3[...]

    bo[...] = tail(bg, br, bw1, bb1, bw2, bb2, bw3, bb3)
    ao[...] = tail(ag, ar, aw1, ab1, aw2, ab2, aw3, ab3)
    to[...] = tail(tg, tr, tw1, tb1, tw2, tb2, tw3, tb3)


def _mlp(n, bg, ag, tg, br, ar, tr, *weights):
    grid = n // _ROWS
    gspec = pl.BlockSpec((_ROWS, _D), lambda i: (i, 0))
    rspec = pl.BlockSpec((_ROWS, 1), lambda i: (i, 0))
    w1spec = pl.BlockSpec((1, _D), lambda i: (0, 0))
    w2spec = pl.BlockSpec((_D, _D), lambda i: (0, 0))
    w3spec = pl.BlockSpec((_D, _NOUT), lambda i: (0, 0))
    b3spec = pl.BlockSpec((1, _NOUT), lambda i: (0, 0))
    ospec = pl.BlockSpec((_ROWS, _NOUT), lambda i: (i, 0))
    tspec = [w1spec, w1spec, w2spec, w1spec, w3spec, b3spec]
    return pl.pallas_call(
        _mlp_body,
        grid=(grid,),
        in_specs=[gspec, gspec, gspec, rspec, rspec, rspec] + tspec * 3,
        out_specs=[ospec, ospec, ospec],
        out_shape=[jax.ShapeDtypeStruct((n, _NOUT), jnp.float32)] * 3,
    )(bg, ag, tg, br, ar, tr, *weights)


def kernel(x_atom, bond_idx, angle_idx, torsion_idx, bond_repr, angle_repr,
           torsion_repr, bond_W1, bond_b1, bond_W2, bond_b2, bond_W3, bond_b3,
           angle_W1, angle_b1, angle_W2, angle_b2, angle_W3, angle_b3,
           torsion_W1, torsion_b1, torsion_W2, torsion_b2, torsion_W3,
           torsion_b3):
    n = bond_idx.shape[0]
    span = _NW * _K * 4
    npad = -(-n // span) * span
    pad = npad - n
    c_per_w = npad // _NW

    w_all = jnp.stack([
        jnp.concatenate([bond_W1[:_D], bond_W1[_D:2 * _D]], axis=1),
        jnp.concatenate([angle_W1[:_D], angle_W1[2 * _D:3 * _D]], axis=1),
        jnp.concatenate([angle_W1[_D:2 * _D]] * 2, axis=1),
        jnp.concatenate([torsion_W1[:_D], torsion_W1[_D:2 * _D]], axis=1),
        jnp.concatenate([torsion_W1[2 * _D:3 * _D],
                         torsion_W1[3 * _D:4 * _D]], axis=1)], axis=0)
    table = _project(x_atom, w_all).reshape(5 * n, _D)

    def prep(idx, cols_offs, kk):
        parts = [jnp.pad(idx[:, c].astype(jnp.int32), (0, pad)) + off * n
                 for c, off in cols_offs]
        x = jnp.stack(parts, axis=0).reshape(len(parts), _NW,
                                             c_per_w // kk, kk)
        return x.transpose(1, 2, 0, 3).reshape(-1)

    bond_cidx = prep(bond_idx, [(0, 0), (1, 0)], 2 * _K)
    angle_cidx = prep(angle_idx, [(0, 1), (1, 2), (2, 1)], 2 * _K)
    torsion_cidx = prep(torsion_idx, [(0, 3), (1, 3), (2, 4), (3, 4)], _K)

    bg = _sc_gather(npad, 2, 2, 2 * _K, _bond_rows)(table, bond_cidx)
    ag = _sc_gather(npad, 3, 2, 2 * _K, _angle_rows)(table, angle_cidx)
    tg = _sc_gather(npad, 4, 2, _K, _torsion_rows)(table, torsion_cidx)

    zeros_h = jnp.zeros((_H, _H), jnp.float32)

    def dup1(v):
        return jnp.concatenate([v.reshape(1, -1)] * 2, axis=1)

    def blkdiag(w2a, w2b):
        return jnp.concatenate(
            [jnp.concatenate([w2a, zeros_h], axis=1),
             jnp.concatenate([zeros_h, w2b], axis=1)], axis=0)

    wtail = (
        dup1(bond_W1[2 * _D]), dup1(bond_b1),
        blkdiag(bond_W2, bond_W2), dup1(bond_b2),
        jnp.concatenate([bond_W3, bond_W3], axis=0),
        (2.0 * bond_b3).reshape(1, _NOUT),
        dup1(angle_W1[3 * _D]), dup1(angle_b1),
        blkdiag(angle_W2, angle_W2), dup1(angle_b2),
        jnp.concatenate([angle_W3, angle_W3], axis=0),
        (2.0 * angle_b3).reshape(1, _NOUT),
        dup1(torsion_W1[4 * _D]), dup1(torsion_b1),
        blkdiag(torsion_W2, zeros_h),
        jnp.concatenate([torsion_b2.reshape(1, _H),
                         jnp.zeros((1, _H), jnp.float32)], axis=1),
        jnp.concatenate([torsion_W3, jnp.zeros((_H, _NOUT), jnp.float32)],
                        axis=0),
        torsion_b3.reshape(1, _NOUT),
    )

    return tuple(_mlp(n, bg, ag, tg, bond_repr, angle_repr, torsion_repr,
                      *wtail))


# speedup vs baseline: 1.0012x; 1.0012x over previous
import functools

import jax
import jax.numpy as jnp
from jax import lax
from jax.experimental import pallas as pl
from jax.experimental.pallas import tpu as pltpu
from jax.experimental.pallas import tpu_sc as plsc

_H = 64
_D = 128
_NOUT = 10
_K = 56
_NW = 32
_ROWS = 1000


def _proj_body(x_ref, w_ref, t_ref):
    x = x_ref[...]
    for q in range(5):
        t_ref[q, :, :] = jnp.dot(x, w_ref[q, :, :],
                                 preferred_element_type=jnp.float32)


def _project(x_atom, w):
    n = x_atom.shape[0]
    grid = n // _ROWS
    return pl.pallas_call(
        _proj_body,
        grid=(grid,),
        in_specs=[pl.BlockSpec((_ROWS, _D), lambda i: (i, 0)),
                  pl.BlockSpec((5, _D, _D), lambda i: (0, 0, 0))],
        out_specs=pl.BlockSpec((5, _ROWS, _D), lambda i: (0, i, 0)),
        out_shape=jax.ShapeDtypeStruct((5, n, _D), jnp.float32),
    )(x_atom, w)


def _sc_gather(npad, n_tab, nbuf, kk, valu_row_factory):
    c_per_w = npad // _NW
    nchunk = c_per_w // kk
    n_grp = nchunk // nbuf
    nc = plsc.get_sparse_core_info().num_cores
    seg = kk * n_tab

    scratch = ([pltpu.VMEM((c_per_w * n_tab,), jnp.int32)]
               + [pltpu.VMEM((seg, _D), jnp.float32)] * nbuf
               + [pltpu.VMEM((kk, _D), jnp.float32)] * nbuf
               + [pltpu.SemaphoreType.DMA] * (2 * nbuf))

    @functools.partial(
        pl.kernel, mesh=plsc.VectorSubcoreMesh(core_axis_name="c",
                                               subcore_axis_name="s"),
        out_type=jax.ShapeDtypeStruct((npad, _D), jnp.float32),
        scratch_types=scratch)
    def k(table, cidx_h, out_h, *scr):
        i_all = scr[0]
        r_v = scr[1:1 + nbuf]
        ov_v = scr[1 + nbuf:1 + 2 * nbuf]
        gsem = scr[1 + 2 * nbuf:1 + 3 * nbuf]
        osem = scr[1 + 3 * nbuf:1 + 4 * nbuf]
        wid = lax.axis_index("s") * nc + lax.axis_index("c")
        base = wid * c_per_w
        valu_row = valu_row_factory(r_v, ov_v, kk)

        pltpu.sync_copy(cidx_h.at[pl.ds(base * n_tab, c_per_w * n_tab)],
                        i_all)

        def issue(c, slot):
            for q in range(n_tab):
                pltpu.async_copy(
                    table.at[i_all.at[pl.ds(c * seg + q * kk, kk)]],
                    r_v[slot].at[pl.ds(q * kk, kk)], gsem[slot])

        def body(g, carry):
            for slot in range(nbuf):
                c = nbuf * g + slot
                off = base + c * kk
                for q in range(n_tab):
                    pltpu.make_async_copy(
                        table.at[i_all.at[pl.ds(c * seg + q * kk, kk)]],
                        r_v[slot].at[pl.ds(q * kk, kk)], gsem[slot]).wait()

                @pl.when(g > 0)
                def _():
                    pltpu.make_async_copy(ov_v[slot],
                                          out_h.at[pl.ds(off, kk)],
                                          osem[slot]).wait()

                def row(i, carry2):
                    valu_row(i, slot)
                    return carry2

                lax.fori_loop(0, kk, row, 0, unroll=8)
                pltpu.async_copy(ov_v[slot], out_h.at[pl.ds(off, kk)],
                                 osem[slot])

                @pl.when(g < n_grp - 1)
                def _():
                    issue(c + nbuf, slot)
            return carry

        for slot in range(nbuf):
            issue(slot, slot)
        lax.fori_loop(0, n_grp, body, 0)
        for slot in range(nbuf):
            pltpu.make_async_copy(ov_v[slot], out_h.at[pl.ds(base, kk)],
                                  osem[slot]).wait()

    return k


def _bond_rows(r_v, ov_v, kk):
    def valu_row(i, slot):
        q = r_v[slot]
        ov = ov_v[slot]
        for j in range(4):
            lo = pl.ds(j * 16, 16)
            hi = pl.ds(_H + j * 16, 16)
            ov[i, lo] = q[i, lo] + q[kk + i, hi]
            ov[i, hi] = q[kk + i, lo] + q[i, hi]
    return valu_row


def _angle_rows(r_v, ov_v, kk):
    def valu_row(i, slot):
        q = r_v[slot]
        ov = ov_v[slot]
        for j in range(4):
            lo = pl.ds(j * 16, 16)
            hi = pl.ds(_H + j * 16, 16)
            mid = q[kk + i, lo]
            ov[i, lo] = q[i, lo] + mid + q[2 * kk + i, hi]
            ov[i, hi] = q[2 * kk + i, lo] + mid + q[i, hi]
    return valu_row


def _torsion_rows(r_v, ov_v, kk):
    def valu_row(i, slot):
        q = r_v[slot]
        ov = ov_v[slot]
        for j in range(4):
            lo = pl.ds(j * 16, 16)
            hi = pl.ds(_H + j * 16, 16)
            g = ((q[i, lo] + q[kk + i, hi])
                 + (q[2 * kk + i, lo] + q[3 * kk + i, hi]))
            ov[i, lo] = g
            ov[i, hi] = g
    return valu_row


def _mlp_body(bg, ag, tg, br, ar, tr,
              bw1, bb1, bw2, bb2, bw3, bb3,
              aw1, ab1, aw2, ab2, aw3, ab3,
              tw1, tb1, tw2, tb2, tw3, tb3,
              bo, ao, to):
    def tail(g, rep, w1, b1, w2, b2, w3, b3):
        h = jax.nn.relu(g[...] + rep[...] * w1[...] + b1[...])
        h = jax.nn.relu(jnp.dot(h, w2[...], preferred_element_type=jnp.float32)
                        + b2[...])
        return jnp.dot(h, w3[...], preferred_element_type=jnp.float32) + b3[...]

    bo[...] = tail(bg, br, bw1, bb1, bw2, bb2, bw3, bb3)
    ao[...] = tail(ag, ar, aw1, ab1, aw2, ab2, aw3, ab3)
    to[...] = tail(tg, tr, tw1, tb1, tw2, tb2, tw3, tb3)


def _mlp(n, bg, ag, tg, br, ar, tr, *weights):
    grid = n // _ROWS
    gspec = pl.BlockSpec((_ROWS, _D), lambda i: (i, 0))
    rspec = pl.BlockSpec((_ROWS, 1), lambda i: (i, 0))
    w1spec = pl.BlockSpec((1, _D), lambda i: (0, 0))
    w2spec = pl.BlockSpec((_D, _D), lambda i: (0, 0))
    w3spec = pl.BlockSpec((_D, _NOUT), lambda i: (0, 0))
    b3spec = pl.BlockSpec((1, _NOUT), lambda i: (0, 0))
    ospec = pl.BlockSpec((_ROWS, _NOUT), lambda i: (i, 0))
    tspec = [w1spec, w1spec, w2spec, w1spec, w3spec, b3spec]
    return pl.pallas_call(
        _mlp_body,
        grid=(grid,),
        in_specs=[gspec, gspec, gspec, rspec, rspec, rspec] + tspec * 3,
        out_specs=[ospec, ospec, ospec],
        out_shape=[jax.ShapeDtypeStruct((n, _NOUT), jnp.float32)] * 3,
    )(bg, ag, tg, br, ar, tr, *weights)


def kernel(x_atom, bond_idx, angle_idx, torsion_idx, bond_repr, angle_repr,
           torsion_repr, bond_W1, bond_b1, bond_W2, bond_b2, bond_W3, bond_b3,
           angle_W1, angle_b1, angle_W2, angle_b2, angle_W3, angle_b3,
           torsion_W1, torsion_b1, torsion_W2, torsion_b2, torsion_W3,
           torsion_b3):
    n = bond_idx.shape[0]
    span = _NW * _K * 4
    npad = -(-n // span) * span
    pad = npad - n
    c_per_w = npad // _NW

    w_all = jnp.stack([
        jnp.concatenate([bond_W1[:_D], bond_W1[_D:2 * _D]], axis=1),
        jnp.concatenate([angle_W1[:_D], angle_W1[2 * _D:3 * _D]], axis=1),
        jnp.concatenate([angle_W1[_D:2 * _D]] * 2, axis=1),
        jnp.concatenate([torsion_W1[:_D], torsion_W1[_D:2 * _D]], axis=1),
        jnp.concatenate([torsion_W1[2 * _D:3 * _D],
                         torsion_W1[3 * _D:4 * _D]], axis=1)], axis=0)
    table = _project(x_atom, w_all).reshape(5 * n, _D)

    def prep(idx, cols_offs, kk):
        parts = [jnp.pad(idx[:, c].astype(jnp.int32), (0, pad)) + off * n
                 for c, off in cols_offs]
        x = jnp.stack(parts, axis=0).reshape(len(parts), _NW,
                                             c_per_w // kk, kk)
        return x.transpose(1, 2, 0, 3).reshape(-1)

    bond_cidx = prep(bond_idx, [(0, 0), (1, 0)], _K)
    angle_cidx = prep(angle_idx, [(0, 1), (1, 2), (2, 1)], _K)
    torsion_cidx = prep(torsion_idx, [(0, 3), (1, 3), (2, 4), (3, 4)], _K)

    bg = _sc_gather(npad, 2, 4, _K, _bond_rows)(table, bond_cidx)
    ag = _sc_gather(npad, 3, 4, _K, _angle_rows)(table, angle_cidx)
    tg = _sc_gather(npad, 4, 2, _K, _torsion_rows)(table, torsion_cidx)

    zeros_h = jnp.zeros((_H, _H), jnp.float32)

    def dup1(v):
        return jnp.concatenate([v.reshape(1, -1)] * 2, axis=1)

    def blkdiag(w2a, w2b):
        return jnp.concatenate(
            [jnp.concatenate([w2a, zeros_h], axis=1),
             jnp.concatenate([zeros_h, w2b], axis=1)], axis=0)

    wtail = (
        dup1(bond_W1[2 * _D]), dup1(bond_b1),
        blkdiag(bond_W2, bond_W2), dup1(bond_b2),
        jnp.concatenate([bond_W3, bond_W3], axis=0),
        (2.0 * bond_b3).reshape(1, _NOUT),
        dup1(angle_W1[3 * _D]), dup1(angle_b1),
        blkdiag(angle_W2, angle_W2), dup1(angle_b2),
        jnp.concatenate([angle_W3, angle_W3], axis=0),
        (2.0 * angle_b3).reshape(1, _NOUT),
        dup1(torsion_W1[4 * _D]), dup1(torsion_b1),
        blkdiag(torsion_W2, zeros_h),
        jnp.concatenate([torsion_b2.reshape(1, _H),
                         jnp.zeros((1, _H), jnp.float32)], axis=1),
        jnp.concatenate([torsion_W3, jnp.zeros((_H, _NOUT), jnp.float32)],
                        axis=0),
        torsion_b3.reshape(1, _NOUT),
    )

    return tuple(_mlp(n, bg, ag, tg, bond_repr, angle_repr, torsion_repr,
                      *wtail))


# speedup vs baseline: 1.0245x; 1.0232x over previous
import functools

import jax
import jax.numpy as jnp
from jax import lax
from jax.experimental import pallas as pl
from jax.experimental.pallas import tpu as pltpu
from jax.experimental.pallas import tpu_sc as plsc

_H = 64
_D = 128
_NOUT = 10
_K = 56
_NW = 32
_ROWS = 1000


def _proj_body(x_ref, wb_ref, wa13_ref, wa2_ref, wt01_ref, wt23_ref,
               tb_ref, ta13_ref, ta2_ref, tt01_ref, tt23_ref):
    x = x_ref[...]
    tb_ref[...] = jnp.dot(x, wb_ref[...], preferred_element_type=jnp.float32)
    ta13_ref[...] = jnp.dot(x, wa13_ref[...], preferred_element_type=jnp.float32)
    ta2_ref[...] = jnp.dot(x, wa2_ref[...], preferred_element_type=jnp.float32)
    tt01_ref[...] = jnp.dot(x, wt01_ref[...], preferred_element_type=jnp.float32)
    tt23_ref[...] = jnp.dot(x, wt23_ref[...], preferred_element_type=jnp.float32)


def _project(x_atom, *ws):
    n = x_atom.shape[0]
    grid = n // _ROWS
    rowspec = pl.BlockSpec((_ROWS, _D), lambda i: (i, 0))
    wspec = pl.BlockSpec((_D, _D), lambda i: (0, 0))
    return pl.pallas_call(
        _proj_body,
        grid=(grid,),
        in_specs=[rowspec] + [wspec] * 5,
        out_specs=[rowspec] * 5,
        out_shape=[jax.ShapeDtypeStruct((n, _D), jnp.float32)] * 5,
    )(x_atom, *ws)


def _sc_gather(npad, n_tab, kk, nbuf, valu_row_factory):
    c_per_w = npad // _NW
    nchunk = c_per_w // kk
    n_grp = nchunk // nbuf
    nc = plsc.get_sparse_core_info().num_cores

    scratch = ([pltpu.VMEM((c_per_w,), jnp.int32)] * n_tab
               + [pltpu.VMEM((kk, _D), jnp.float32)] * (nbuf * n_tab)
               + [pltpu.VMEM((kk, _D), jnp.float32)] * nbuf
               + [pltpu.SemaphoreType.DMA] * (2 * nbuf))

    @functools.partial(
        pl.kernel, mesh=plsc.VectorSubcoreMesh(core_axis_name="c",
                                               subcore_axis_name="s"),
        out_type=jax.ShapeDtypeStruct((npad, _D), jnp.float32),
        scratch_types=scratch)
    def k(*args):
        tables = args[:n_tab]
        idx_hs = args[n_tab:2 * n_tab]
        out_h = args[2 * n_tab]
        scr = args[2 * n_tab + 1:]
        i_all = scr[:n_tab]
        r_v = tuple(scr[n_tab + s * n_tab: n_tab + (s + 1) * n_tab]
                    for s in range(nbuf))
        rest = scr[n_tab + nbuf * n_tab:]
        ov_v = rest[:nbuf]
        gsem = rest[nbuf:2 * nbuf]
        osem = rest[2 * nbuf:3 * nbuf]
        wid = lax.axis_index("s") * nc + lax.axis_index("c")
        base = wid * c_per_w
        valu_row = valu_row_factory(r_v, ov_v)

        for q in range(n_tab):
            pltpu.sync_copy(idx_hs[q].at[pl.ds(base, c_per_w)], i_all[q])

        def issue(c, slot):
            for q in range(n_tab):
                pltpu.async_copy(
                    tables[q].at[i_all[q].at[pl.ds(c * kk, kk)]],
                    r_v[slot][q], gsem[slot])

        def body(g, carry):
            for slot in range(nbuf):
                c = nbuf * g + slot
                off = base + c * kk
                for q in range(n_tab):
                    pltpu.make_async_copy(
                        tables[q].at[i_all[q].at[pl.ds(c * kk, kk)]],
                        r_v[slot][q], gsem[slot]).wait()

                @pl.when(g > 0)
                def _():
                    pltpu.make_async_copy(ov_v[slot],
                                          out_h.at[pl.ds(off, kk)],
                                          osem[slot]).wait()

                def row(i, carry2):
                    valu_row(i, slot)
                    return carry2

                lax.fori_loop(0, kk, row, 0, unroll=4)
                pltpu.async_copy(ov_v[slot], out_h.at[pl.ds(off, kk)],
                                 osem[slot])

                @pl.when(g < n_grp - 1)
                def _():
                    issue(c + nbuf, slot)
            return carry

        for slot in range(nbuf):
            issue(slot, slot)
        lax.fori_loop(0, n_grp, body, 0)
        for slot in range(nbuf):
            pltpu.make_async_copy(ov_v[slot], out_h.at[pl.ds(base, kk)],
                                  osem[slot]).wait()

    return k


def _bond_rows(r_v, ov_v):
    def valu_row(i, slot):
        r0, r1 = r_v[slot][0], r_v[slot][1]
        ov = ov_v[slot]
        for j in range(4):
            lo = pl.ds(j * 16, 16)
            hi = pl.ds(_H + j * 16, 16)
            ov[i, lo] = r0[i, lo] + r1[i, hi]
            ov[i, hi] = r1[i, lo] + r0[i, hi]
    return valu_row


def _angle_rows(r_v, ov_v):
    def valu_row(i, slot):
        u0, a2v, u2 = r_v[slot][0], r_v[slot][1], r_v[slot][2]
        ov = ov_v[slot]
        for j in range(4):
            lo = pl.ds(j * 16, 16)
            hi = pl.ds(_H + j * 16, 16)
            mid = a2v[i, lo]
            ov[i, lo] = u0[i, lo] + mid + u2[i, hi]
            ov[i, hi] = u2[i, lo] + mid + u0[i, hi]
    return valu_row


def _torsion_rows(r_v, ov_v):
    def valu_row(i, slot):
        r0, r1, r2, r3 = r_v[slot]
        ov = ov_v[slot]
        for j in range(4):
            lo = pl.ds(j * 16, 16)
            hi = pl.ds(_H + j * 16, 16)
            g = (r0[i, lo] + r1[i, hi]) + (r2[i, lo] + r3[i, hi])
            ov[i, lo] = g
            ov[i, hi] = g
    return valu_row


def _mlp_body(g_ref, r_ref, w1_ref, b1_ref, w2_ref, b2_ref, w3_ref, b3_ref,
              o_ref):
    h = jax.nn.relu(g_ref[...] + r_ref[...] * w1_ref[...] + b1_ref[...])
    h = jax.nn.relu(jnp.dot(h, w2_ref[...], preferred_element_type=jnp.float32)
                    + b2_ref[...])
    o_ref[...] = (jnp.dot(h, w3_ref[...], preferred_element_type=jnp.float32)
                  + b3_ref[...])


def _mlp(n, g, rep, w1, b1, w2, b2, w3, b3):
    grid = n // _ROWS
    return pl.pallas_call(
        _mlp_body,
        grid=(grid,),
        in_specs=[pl.BlockSpec((_ROWS, _D), lambda i: (i, 0)),
                  pl.BlockSpec((_ROWS, 1), lambda i: (i, 0)),
                  pl.BlockSpec((1, _D), lambda i: (0, 0)),
                  pl.BlockSpec((1, _D), lambda i: (0, 0)),
                  pl.BlockSpec((_D, _D), lambda i: (0, 0)),
                  pl.BlockSpec((1, _D), lambda i: (0, 0)),
                  pl.BlockSpec((_D, _NOUT), lambda i: (0, 0)),
                  pl.BlockSpec((1, _NOUT), lambda i: (0, 0))],
        out_specs=pl.BlockSpec((_ROWS, _NOUT), lambda i: (i, 0)),
        out_shape=jax.ShapeDtypeStruct((n, _NOUT), jnp.float32),
    )(g, rep, w1, b1, w2, b2, w3, b3)


def kernel(x_atom, bond_idx, angle_idx, torsion_idx, bond_repr, angle_repr,
           torsion_repr, bond_W1, bond_b1, bond_W2, bond_b2, bond_W3, bond_b3,
           angle_W1, angle_b1, angle_W2, angle_b2, angle_W3, angle_b3,
           torsion_W1, torsion_b1, torsion_W2, torsion_b2, torsion_W3,
           torsion_b3):
    n = bond_idx.shape[0]
    span = _NW * _K * 4
    npad = -(-n // span) * span
    pad = npad - n

    tb, ta13, ta2, tt01, tt23 = _project(
        x_atom,
        jnp.concatenate([bond_W1[:_D], bond_W1[_D:2 * _D]], axis=1),
        jnp.concatenate([angle_W1[:_D], angle_W1[2 * _D:3 * _D]], axis=1),
        jnp.concatenate([angle_W1[_D:2 * _D]] * 2, axis=1),
        jnp.concatenate([torsion_W1[:_D], torsion_W1[_D:2 * _D]], axis=1),
        jnp.concatenate([torsion_W1[2 * _D:3 * _D],
                         torsion_W1[3 * _D:4 * _D]], axis=1))

    def prep(idx, col):
        return jnp.pad(idx[:, col].astype(jnp.int32), (0, pad))

    b0, b1i = prep(bond_idx, 0), prep(bond_idx, 1)
    a0, a1i, a2i = (prep(angle_idx, c) for c in range(3))
    t0, t1i, t2i, t3i = (prep(torsion_idx, c) for c in range(4))

    bg = _sc_gather(npad, 2, _K, 4, _bond_rows)(tb, tb, b0, b1i)
    ag = _sc_gather(npad, 3, _K, 4, _angle_rows)(ta13, ta2, ta13, a0, a1i, a2i)
    tg = _sc_gather(npad, 4, _K, 2, _torsion_rows)(tt01, tt01, tt23, tt23,
                                                   t0, t1i, t2i, t3i)

    zeros_h = jnp.zeros((_H, _H), jnp.float32)

    def dup1(v):
        return jnp.concatenate([v.reshape(1, -1)] * 2, axis=1)

    def blkdiag(w2a, w2b):
        return jnp.concatenate(
            [jnp.concatenate([w2a, zeros_h], axis=1),
             jnp.concatenate([zeros_h, w2b], axis=1)], axis=0)

    bo = _mlp(n, bg, bond_repr, dup1(bond_W1[2 * _D]), dup1(bond_b1),
              blkdiag(bond_W2, bond_W2), dup1(bond_b2),
              jnp.concatenate([bond_W3, bond_W3], axis=0),
              (2.0 * bond_b3).reshape(1, _NOUT))
    ao = _mlp(n, ag, angle_repr, dup1(angle_W1[3 * _D]), dup1(angle_b1),
              blkdiag(angle_W2, angle_W2), dup1(angle_b2),
              jnp.concatenate([angle_W3, angle_W3], axis=0),
              (2.0 * angle_b3).reshape(1, _NOUT))
    to = _mlp(n, tg, torsion_repr, dup1(torsion_W1[4 * _D]), dup1(torsion_b1),
              blkdiag(torsion_W2, zeros_h),
              jnp.concatenate([torsion_b2.reshape(1, _H),
                               jnp.zeros((1, _H), jnp.float32)], axis=1),
              jnp.concatenate([torsion_W3, jnp.zeros((_H, _NOUT),
                                                     jnp.float32)], axis=0),
              torsion_b3.reshape(1, _NOUT))

    return (bo, ao, to)
